# baseline (speedup 1.0000x reference)
import functools

import jax
import jax.numpy as jnp
from jax import lax
from jax.experimental import pallas as pl
from jax.experimental.pallas import tpu as pltpu
from jax.experimental.pallas import tpu_sc as plsc

ALPHA = 0.25
GAMMA = 2.0

_NC = 2
_NS = 16
_NW = _NC * _NS
_LANES = 16

_TOTAL_ROWS = 32768
_ROWS = 2048
_R_SC = 10240
_CH = 16
_SC_CH = 8192

_LOG1P_COEF = (
    0.014201727447196227, -0.06658471287014109, 0.149430702293233,
    -0.23514648274176575, 0.3311199413645243, -0.4998718500618637,
    0.9999987613784038,
)


def _focal_terms(x, t, use_poly_log1p):
    ax = jnp.abs(x)
    e = jnp.exp(-ax)
    if use_poly_log1p:
        r = jnp.full_like(e, _LOG1P_COEF[0])
        for c in _LOG1P_COEF[1:]:
            r = r * e + c
        sp = e * r
    else:
        sp = jnp.log1p(e)
    ce = jnp.maximum(x, 0.0) - x * t + sp
    numer = jnp.where(x >= 0.0, jnp.ones_like(e), e)
    p = numer / (1.0 + e)
    q = t + p * (1.0 - 2.0 * t)
    at = 0.75 - 0.5 * t
    val = ce * (q * q) * at
    val = jnp.where(ax + t != 0.0, val, 0.0)
    pos = jnp.where(t > 0.5, 1.0, 0.0)
    return val, pos


def _tc_body(pred_ref, true_ref, loss_ref, npos_ref):
    def step(i, carry):
        acc_l, acc_n = carry
        x = pred_ref[pl.ds(i * _CH, _CH), :]
        t = true_ref[pl.ds(i * _CH, _CH), :]
        val, pos = _focal_terms(x, t, use_poly_log1p=False)
        return acc_l + val, acc_n + pos

    z = jnp.zeros((_CH, 512), jnp.float32)
    acc_l, acc_n = lax.fori_loop(0, _ROWS // _CH, step, (z, z))
    part_loss = jnp.sum(acc_l)
    part_npos = jnp.sum(acc_n)

    @pl.when(pl.program_id(0) == 0)
    def _init():
        loss_ref[0] = 0.0
        npos_ref[0] = 0.0

    loss_ref[0] += part_loss
    npos_ref[0] += part_npos


def _tc_partial(pred2d, true2d, row_off, n_rows):
    grid = n_rows // _ROWS
    blk_off = row_off // _ROWS
    loss_sum, npos = pl.pallas_call(
        _tc_body,
        grid=(grid,),
        in_specs=[
            pl.BlockSpec((_ROWS, 512), lambda i: (i + blk_off, 0)),
            pl.BlockSpec((_ROWS, 512), lambda i: (i + blk_off, 0)),
        ],
        out_specs=[
            pl.BlockSpec(memory_space=pltpu.SMEM),
            pl.BlockSpec(memory_space=pltpu.SMEM),
        ],
        out_shape=[
            jax.ShapeDtypeStruct((1,), jnp.float32),
            jax.ShapeDtypeStruct((1,), jnp.float32),
        ],
    )(pred2d, true2d)
    return loss_sum[0], npos[0]


_SC_CHR = 16


def _make_sc_partial(n_rows):
    per_tile = n_rows // _NW
    n_chunks = per_tile // _SC_CHR
    assert per_tile % _SC_CHR == 0 and n_chunks % 2 == 0

    mesh = plsc.VectorSubcoreMesh(core_axis_name="c", subcore_axis_name="s")

    @functools.partial(
        pl.kernel,
        out_type=[
            jax.ShapeDtypeStruct((_NW, _LANES), jnp.float32),
            jax.ShapeDtypeStruct((_NW, _LANES), jnp.float32),
        ],
        mesh=mesh,
        compiler_params=pltpu.CompilerParams(use_tc_tiling_on_sc=True, skip_device_barrier=True),
        scratch_types=[
            pltpu.VMEM((_SC_CHR, 512), jnp.float32),
            pltpu.VMEM((_SC_CHR, 512), jnp.float32),
            pltpu.VMEM((_SC_CHR, 512), jnp.float32),
            pltpu.VMEM((_SC_CHR, 512), jnp.float32),
            pltpu.VMEM((_LANES,), jnp.float32),
            pltpu.VMEM((_LANES,), jnp.float32),
            pltpu.SemaphoreType.DMA,
            pltpu.SemaphoreType.DMA,
        ],
    )
    def sc_kernel(pred_hbm, true_hbm, loss_out, npos_out,
                  pb0, pb1, tb0, tb1, accl_v, accn_v, sem0, sem1):
        wid = lax.axis_index("s") * _NC + lax.axis_index("c")
        base = wid * per_tile
        pbufs = (pb0, pb1)
        tbufs = (tb0, tb1)
        sems = (sem0, sem1)

        for b in range(2):
            off = base + b * _SC_CHR
            pltpu.async_copy(
                pred_hbm.at[pl.ds(off, _SC_CHR), :], pbufs[b], sems[b])
            pltpu.async_copy(
                true_hbm.at[pl.ds(off, _SC_CHR), :], tbufs[b], sems[b])

        def compute_chunk(pb, tb, acc):
            def step(i, carry):
                acc_l0, acc_n0, acc_l1, acc_n1 = carry
                r = i >> 4
                col = (i & 15) * (2 * _LANES)
                x0 = pb[r, pl.ds(col, _LANES)]
                t0 = tb[r, pl.ds(col, _LANES)]
                x1 = pb[r, pl.ds(col + _LANES, _LANES)]
                t1 = tb[r, pl.ds(col + _LANES, _LANES)]
                val0, pos0 = _focal_terms(x0, t0, use_poly_log1p=True)
                val1, pos1 = _focal_terms(x1, t1, use_poly_log1p=True)
                return (acc_l0 + val0, acc_n0 + pos0,
                        acc_l1 + val1, acc_n1 + pos1)
            n_iters = _SC_CHR * (512 // (2 * _LANES))
            a = lax.fori_loop(0, n_iters, step, (acc[0], acc[1], acc[0] * 0.0, acc[1] * 0.0))
            return a[0] + a[2], a[1] + a[3]

        def outer(j, acc):
            for b in range(2):
                g = 2 * j + b
                off = base + g * _SC_CHR
                pltpu.make_async_copy(
                    pred_hbm.at[pl.ds(off, _SC_CHR), :], pbufs[b], sems[b]).wait()
                pltpu.make_async_copy(
                    true_hbm.at[pl.ds(off, _SC_CHR), :], tbufs[b], sems[b]).wait()
                acc = compute_chunk(pbufs[b], tbufs[b], acc)
                nxt = g + 2

                @pl.when(nxt < n_chunks)
                def _prefetch():
                    noff = base + nxt * _SC_CHR
                    pltpu.async_copy(
                        pred_hbm.at[pl.ds(noff, _SC_CHR), :], pbufs[b], sems[b])
                    pltpu.async_copy(
                        true_hbm.at[pl.ds(noff, _SC_CHR), :], tbufs[b], sems[b])
            return acc

        z = jnp.zeros((_LANES,), jnp.float32)
        acc_l, acc_n = lax.fori_loop(0, n_chunks // 2, outer, (z, z))
        accl_v[...] = acc_l
        accn_v[...] = acc_n
        pltpu.sync_copy(accl_v, loss_out.at[wid])
        pltpu.sync_copy(accn_v, npos_out.at[wid])

    return sc_kernel


def kernel(predicted_foreground_masks, peak_normalized_images):
    pred2d = predicted_foreground_masks.reshape(_TOTAL_ROWS, 512)
    true2d = peak_normalized_images.reshape(_TOTAL_ROWS, 512)

    loss_parts = []
    npos_parts = []

    if _R_SC > 0:
        sc = _make_sc_partial(_R_SC)
        sc_loss, sc_npos = sc(pred2d, true2d)
        loss_parts.append(jnp.sum(sc_loss))
        npos_parts.append(jnp.sum(sc_npos))

    if _R_SC < _TOTAL_ROWS:
        tc_loss, tc_npos = _tc_partial(pred2d, true2d, _R_SC,
                                       _TOTAL_ROWS - _R_SC)
        loss_parts.append(tc_loss)
        npos_parts.append(tc_npos)

    loss_sum = sum(loss_parts)
    npos = sum(npos_parts)
    return loss_sum / jnp.maximum(npos, 1.0)


# speedup vs baseline: 1.1577x; 1.1577x over previous
import functools

import jax
import jax.numpy as jnp
from jax import lax
from jax.experimental import pallas as pl
from jax.experimental.pallas import tpu as pltpu
from jax.experimental.pallas import tpu_sc as plsc

ALPHA = 0.25
GAMMA = 2.0

_NC = 2
_NS = 16
_NW = _NC * _NS
_LANES = 16

_TOTAL_ROWS = 32768
_ROWS = 4096
_R_SC = 8192
_CH = 16
_SC_CH = 8192

_LOG1P_COEF = (
    0.014201727447196227, -0.06658471287014109, 0.149430702293233,
    -0.23514648274176575, 0.3311199413645243, -0.4998718500618637,
    0.9999987613784038,
)


def _focal_terms(x, t, use_poly_log1p):
    ax = jnp.abs(x)
    e = jnp.exp(-ax)
    if use_poly_log1p:
        r = jnp.full_like(e, _LOG1P_COEF[0])
        for c in _LOG1P_COEF[1:]:
            r = r * e + c
        sp = e * r
    else:
        sp = jnp.log1p(e)
    ce = jnp.maximum(x, 0.0) - x * t + sp
    numer = jnp.where(x >= 0.0, jnp.ones_like(e), e)
    p = numer / (1.0 + e)
    q = t + p * (1.0 - 2.0 * t)
    at = 0.75 - 0.5 * t
    val = ce * (q * q) * at
    val = jnp.where(ax + t != 0.0, val, 0.0)
    pos = jnp.where(t > 0.5, 1.0, 0.0)
    return val, pos


def _tc_body(pred_ref, true_ref, loss_ref, npos_ref):
    def step(i, carry):
        acc_l, acc_n = carry
        x = pred_ref[pl.ds(i * _CH, _CH), :]
        t = true_ref[pl.ds(i * _CH, _CH), :]
        val, pos = _focal_terms(x, t, use_poly_log1p=False)
        return acc_l + val, acc_n + pos

    z = jnp.zeros((_CH, 512), jnp.float32)
    acc_l, acc_n = lax.fori_loop(0, _ROWS // _CH, step, (z, z))
    part_loss = jnp.sum(acc_l)
    part_npos = jnp.sum(acc_n)

    @pl.when(pl.program_id(0) == 0)
    def _init():
        loss_ref[0] = 0.0
        npos_ref[0] = 0.0

    loss_ref[0] += part_loss
    npos_ref[0] += part_npos


def _tc_partial(pred2d, true2d, row_off, n_rows):
    grid = n_rows // _ROWS
    blk_off = row_off // _ROWS
    loss_sum, npos = pl.pallas_call(
        _tc_body,
        grid=(grid,),
        in_specs=[
            pl.BlockSpec((_ROWS, 512), lambda i: (i + blk_off, 0)),
            pl.BlockSpec((_ROWS, 512), lambda i: (i + blk_off, 0)),
        ],
        out_specs=[
            pl.BlockSpec(memory_space=pltpu.SMEM),
            pl.BlockSpec(memory_space=pltpu.SMEM),
        ],
        out_shape=[
            jax.ShapeDtypeStruct((1,), jnp.float32),
            jax.ShapeDtypeStruct((1,), jnp.float32),
        ],
    )(pred2d, true2d)
    return loss_sum[0], npos[0]


_SC_CHR = 16


def _make_sc_partial(n_rows):
    per_tile = n_rows // _NW
    n_chunks = per_tile // _SC_CHR
    assert per_tile % _SC_CHR == 0 and n_chunks % 2 == 0

    mesh = plsc.VectorSubcoreMesh(core_axis_name="c", subcore_axis_name="s")

    @functools.partial(
        pl.kernel,
        out_type=[
            jax.ShapeDtypeStruct((_NW, _LANES), jnp.float32),
            jax.ShapeDtypeStruct((_NW, _LANES), jnp.float32),
        ],
        mesh=mesh,
        compiler_params=pltpu.CompilerParams(use_tc_tiling_on_sc=True, skip_device_barrier=True),
        scratch_types=[
            pltpu.VMEM((_SC_CHR, 512), jnp.float32),
            pltpu.VMEM((_SC_CHR, 512), jnp.float32),
            pltpu.VMEM((_SC_CHR, 512), jnp.float32),
            pltpu.VMEM((_SC_CHR, 512), jnp.float32),
            pltpu.VMEM((_LANES,), jnp.float32),
            pltpu.VMEM((_LANES,), jnp.float32),
            pltpu.SemaphoreType.DMA,
            pltpu.SemaphoreType.DMA,
        ],
    )
    def sc_kernel(pred_hbm, true_hbm, loss_out, npos_out,
                  pb0, pb1, tb0, tb1, accl_v, accn_v, sem0, sem1):
        wid = lax.axis_index("s") * _NC + lax.axis_index("c")
        base = wid * per_tile
        pbufs = (pb0, pb1)
        tbufs = (tb0, tb1)
        sems = (sem0, sem1)

        for b in range(2):
            off = base + b * _SC_CHR
            pltpu.async_copy(
                pred_hbm.at[pl.ds(off, _SC_CHR), :], pbufs[b], sems[b])
            pltpu.async_copy(
                true_hbm.at[pl.ds(off, _SC_CHR), :], tbufs[b], sems[b])

        def compute_chunk(pb, tb, acc):
            def step(i, carry):
                acc_l0, acc_n0, acc_l1, acc_n1 = carry
                r = i >> 4
                col = (i & 15) * (2 * _LANES)
                x0 = pb[r, pl.ds(col, _LANES)]
                t0 = tb[r, pl.ds(col, _LANES)]
                x1 = pb[r, pl.ds(col + _LANES, _LANES)]
                t1 = tb[r, pl.ds(col + _LANES, _LANES)]
                val0, pos0 = _focal_terms(x0, t0, use_poly_log1p=True)
                val1, pos1 = _focal_terms(x1, t1, use_poly_log1p=True)
                return (acc_l0 + val0, acc_n0 + pos0,
                        acc_l1 + val1, acc_n1 + pos1)
            n_iters = _SC_CHR * (512 // (2 * _LANES))
            a = lax.fori_loop(0, n_iters, step, (acc[0], acc[1], acc[0] * 0.0, acc[1] * 0.0))
            return a[0] + a[2], a[1] + a[3]

        def outer(j, acc):
            for b in range(2):
                g = 2 * j + b
                off = base + g * _SC_CHR
                pltpu.make_async_copy(
                    pred_hbm.at[pl.ds(off, _SC_CHR), :], pbufs[b], sems[b]).wait()
                pltpu.make_async_copy(
                    true_hbm.at[pl.ds(off, _SC_CHR), :], tbufs[b], sems[b]).wait()
                acc = compute_chunk(pbufs[b], tbufs[b], acc)
                nxt = g + 2

                @pl.when(nxt < n_chunks)
                def _prefetch():
                    noff = base + nxt * _SC_CHR
                    pltpu.async_copy(
                        pred_hbm.at[pl.ds(noff, _SC_CHR), :], pbufs[b], sems[b])
                    pltpu.async_copy(
                        true_hbm.at[pl.ds(noff, _SC_CHR), :], tbufs[b], sems[b])
            return acc

        z = jnp.zeros((_LANES,), jnp.float32)
        acc_l, acc_n = lax.fori_loop(0, n_chunks // 2, outer, (z, z))
        accl_v[...] = acc_l
        accn_v[...] = acc_n
        pltpu.sync_copy(accl_v, loss_out.at[wid])
        pltpu.sync_copy(accn_v, npos_out.at[wid])

    return sc_kernel


def kernel(predicted_foreground_masks, peak_normalized_images):
    pred2d = predicted_foreground_masks.reshape(_TOTAL_ROWS, 512)
    true2d = peak_normalized_images.reshape(_TOTAL_ROWS, 512)

    loss_parts = []
    npos_parts = []

    if _R_SC > 0:
        sc = _make_sc_partial(_R_SC)
        sc_loss, sc_npos = sc(pred2d, true2d)
        loss_parts.append(jnp.sum(sc_loss))
        npos_parts.append(jnp.sum(sc_npos))

    if _R_SC < _TOTAL_ROWS:
        tc_loss, tc_npos = _tc_partial(pred2d, true2d, _R_SC,
                                       _TOTAL_ROWS - _R_SC)
        loss_parts.append(tc_loss)
        npos_parts.append(tc_npos)

    loss_sum = sum(loss_parts)
    npos = sum(npos_parts)
    return loss_sum / jnp.maximum(npos, 1.0)


# speedup vs baseline: 1.1773x; 1.0169x over previous
import functools

import jax
import jax.numpy as jnp
from jax import lax
from jax.experimental import pallas as pl
from jax.experimental.pallas import tpu as pltpu
from jax.experimental.pallas import tpu_sc as plsc

ALPHA = 0.25
GAMMA = 2.0

_NC = 2
_NS = 16
_NW = _NC * _NS
_LANES = 16

_TOTAL_ROWS = 32768
_ROWS = 2048
_R_SC = 8192
_CH = 16
_SC_CH = 8192

_LOG1P_COEF = (
    0.014201727447196227, -0.06658471287014109, 0.149430702293233,
    -0.23514648274176575, 0.3311199413645243, -0.4998718500618637,
    0.9999987613784038,
)


def _focal_terms(x, t, use_poly_log1p):
    ax = jnp.abs(x)
    e = jnp.exp(-ax)
    if use_poly_log1p:
        r = jnp.full_like(e, _LOG1P_COEF[0])
        for c in _LOG1P_COEF[1:]:
            r = r * e + c
        sp = e * r
    else:
        sp = jnp.log1p(e)
    ce = jnp.maximum(x, 0.0) - x * t + sp
    numer = jnp.where(x >= 0.0, jnp.ones_like(e), e)
    p = numer / (1.0 + e)
    q = t + p * (1.0 - 2.0 * t)
    at = 0.75 - 0.5 * t
    val = ce * (q * q) * at
    val = jnp.where(ax + t != 0.0, val, 0.0)
    pos = jnp.where(t > 0.5, 1.0, 0.0)
    return val, pos


def _tc_body(pred_ref, true_ref, loss_ref, npos_ref):
    def step(i, carry):
        acc_l, acc_n = carry
        x = pred_ref[pl.ds(i * _CH, _CH), :]
        t = true_ref[pl.ds(i * _CH, _CH), :]
        val, pos = _focal_terms(x, t, use_poly_log1p=False)
        return acc_l + val, acc_n + pos

    z = jnp.zeros((_CH, 512), jnp.float32)
    acc_l, acc_n = lax.fori_loop(0, _ROWS // _CH, step, (z, z))
    part_loss = jnp.sum(acc_l)
    part_npos = jnp.sum(acc_n)

    @pl.when(pl.program_id(0) == 0)
    def _init():
        loss_ref[0] = 0.0
        npos_ref[0] = 0.0

    loss_ref[0] += part_loss
    npos_ref[0] += part_npos


def _tc_partial(pred2d, true2d, row_off, n_rows):
    grid = n_rows // _ROWS
    blk_off = row_off // _ROWS
    loss_sum, npos = pl.pallas_call(
        _tc_body,
        grid=(grid,),
        in_specs=[
            pl.BlockSpec((_ROWS, 512), lambda i: (i + blk_off, 0)),
            pl.BlockSpec((_ROWS, 512), lambda i: (i + blk_off, 0)),
        ],
        out_specs=[
            pl.BlockSpec(memory_space=pltpu.SMEM),
            pl.BlockSpec(memory_space=pltpu.SMEM),
        ],
        out_shape=[
            jax.ShapeDtypeStruct((1,), jnp.float32),
            jax.ShapeDtypeStruct((1,), jnp.float32),
        ],
    )(pred2d, true2d)
    return loss_sum[0], npos[0]


_SC_CHR = 16


def _make_sc_partial(n_rows):
    per_tile = n_rows // _NW
    n_chunks = per_tile // _SC_CHR
    assert per_tile % _SC_CHR == 0 and n_chunks % 2 == 0

    mesh = plsc.VectorSubcoreMesh(core_axis_name="c", subcore_axis_name="s")

    @functools.partial(
        pl.kernel,
        out_type=[
            jax.ShapeDtypeStruct((_NW, _LANES), jnp.float32),
            jax.ShapeDtypeStruct((_NW, _LANES), jnp.float32),
        ],
        mesh=mesh,
        compiler_params=pltpu.CompilerParams(use_tc_tiling_on_sc=True, skip_device_barrier=True),
        scratch_types=[
            pltpu.VMEM((_SC_CHR, 512), jnp.float32),
            pltpu.VMEM((_SC_CHR, 512), jnp.float32),
            pltpu.VMEM((_SC_CHR, 512), jnp.float32),
            pltpu.VMEM((_SC_CHR, 512), jnp.float32),
            pltpu.VMEM((_LANES,), jnp.float32),
            pltpu.VMEM((_LANES,), jnp.float32),
            pltpu.SemaphoreType.DMA,
            pltpu.SemaphoreType.DMA,
        ],
    )
    def sc_kernel(pred_hbm, true_hbm, loss_out, npos_out,
                  pb0, pb1, tb0, tb1, accl_v, accn_v, sem0, sem1):
        wid = lax.axis_index("s") * _NC + lax.axis_index("c")
        base = wid * per_tile
        pbufs = (pb0, pb1)
        tbufs = (tb0, tb1)
        sems = (sem0, sem1)

        for b in range(2):
            off = base + b * _SC_CHR
            pltpu.async_copy(
                pred_hbm.at[pl.ds(off, _SC_CHR), :], pbufs[b], sems[b])
            pltpu.async_copy(
                true_hbm.at[pl.ds(off, _SC_CHR), :], tbufs[b], sems[b])

        def compute_chunk(pb, tb, acc):
            def step(i, carry):
                acc_l0, acc_n0, acc_l1, acc_n1 = carry
                r = i >> 4
                col = (i & 15) * (2 * _LANES)
                x0 = pb[r, pl.ds(col, _LANES)]
                t0 = tb[r, pl.ds(col, _LANES)]
                x1 = pb[r, pl.ds(col + _LANES, _LANES)]
                t1 = tb[r, pl.ds(col + _LANES, _LANES)]
                val0, pos0 = _focal_terms(x0, t0, use_poly_log1p=True)
                val1, pos1 = _focal_terms(x1, t1, use_poly_log1p=True)
                return (acc_l0 + val0, acc_n0 + pos0,
                        acc_l1 + val1, acc_n1 + pos1)
            n_iters = _SC_CHR * (512 // (2 * _LANES))
            a = lax.fori_loop(0, n_iters, step, (acc[0], acc[1], acc[0] * 0.0, acc[1] * 0.0))
            return a[0] + a[2], a[1] + a[3]

        def outer(j, acc):
            for b in range(2):
                g = 2 * j + b
                off = base + g * _SC_CHR
                pltpu.make_async_copy(
                    pred_hbm.at[pl.ds(off, _SC_CHR), :], pbufs[b], sems[b]).wait()
                pltpu.make_async_copy(
                    true_hbm.at[pl.ds(off, _SC_CHR), :], tbufs[b], sems[b]).wait()
                acc = compute_chunk(pbufs[b], tbufs[b], acc)
                nxt = g + 2

                @pl.when(nxt < n_chunks)
                def _prefetch():
                    noff = base + nxt * _SC_CHR
                    pltpu.async_copy(
                        pred_hbm.at[pl.ds(noff, _SC_CHR), :], pbufs[b], sems[b])
                    pltpu.async_copy(
                        true_hbm.at[pl.ds(noff, _SC_CHR), :], tbufs[b], sems[b])
            return acc

        z = jnp.zeros((_LANES,), jnp.float32)
        acc_l, acc_n = lax.fori_loop(0, n_chunks // 2, outer, (z, z))
        accl_v[...] = acc_l
        accn_v[...] = acc_n
        pltpu.sync_copy(accl_v, loss_out.at[wid])
        pltpu.sync_copy(accn_v, npos_out.at[wid])

    return sc_kernel


def kernel(predicted_foreground_masks, peak_normalized_images):
    pred2d = predicted_foreground_masks.reshape(_TOTAL_ROWS, 512)
    true2d = peak_normalized_images.reshape(_TOTAL_ROWS, 512)

    loss_parts = []
    npos_parts = []

    if _R_SC > 0:
        sc = _make_sc_partial(_R_SC)
        sc_loss, sc_npos = sc(pred2d, true2d)
        loss_parts.append(jnp.sum(sc_loss))
        npos_parts.append(jnp.sum(sc_npos))

    if _R_SC < _TOTAL_ROWS:
        tc_loss, tc_npos = _tc_partial(pred2d, true2d, _R_SC,
                                       _TOTAL_ROWS - _R_SC)
        loss_parts.append(tc_loss)
        npos_parts.append(tc_npos)

    loss_sum = sum(loss_parts)
    npos = sum(npos_parts)
    return loss_sum / jnp.maximum(npos, 1.0)
